# baseline (speedup 1.0000x reference)
import jax
import jax.numpy as jnp
from jax import lax
from jax.experimental import pallas as pl
from jax.experimental.pallas import tpu as pltpu
from jax.experimental.pallas import tpu_sc as plsc

L = 16
NW = 32
NT = 4096 * 24
S = 128
SSC = 32
CPW = NT // NW
GPW = CPW // L
SCHUNK = 4
NCH = SSC // SCHUNK
QSTG = 1024
NTC = S - SSC
CB = 3072


def _tc_body(qT_ref, u_ref, taus_ref, out_ref):
    t = [taus_ref[j] for j in range(9)]
    iv = [1.0 / (t[j + 1] - t[j] + 1e-12) for j in range(8)]
    cum = qT_ref[0:1, :]
    first = cum
    svecs = []
    for j in range(8):
        nxt = jnp.maximum(cum, qT_ref[j + 1:j + 2, :])
        svecs.append((nxt - cum) * iv[j])
        cum = nxt
    a = first - svecs[0] * t[0]
    b = svecs[0]
    ub = u_ref[...]
    terms = [a + b * ub]
    for j in range(1, 8):
        dj = svecs[j] - svecs[j - 1]
        terms.append(dj * jnp.maximum(ub - t[j], 0.0))
    while len(terms) > 1:
        terms = [terms[i] + terms[i + 1]
                 for i in range(0, len(terms) - 1, 2)] + (
                     [terms[-1]] if len(terms) % 2 else [])
    out_ref[...] = jnp.maximum(terms[0], 0.0)


def _sc_body(qT, u2, tsp, iv, out,
             qbuf, coef, tbuf, ibuf,
             ub0, ub1, ob0, ob1, us0, us1, os0, os1):
    nc = 2
    wid = lax.axis_index("s") * nc + lax.axis_index("c")
    base = wid * CPW

    pltpu.sync_copy(tsp, tbuf)
    pltpu.sync_copy(iv, ibuf)

    ivecs = [ibuf[j, :] for j in range(8)]
    t0 = tbuf[0, :]
    tvecs = [tbuf[j, :] for j in range(1, 8)]

    for p in range(CPW // QSTG):
        pltpu.sync_copy(qT.at[:, pl.ds(base + p * QSTG, QSTG)], qbuf)

        @pl.loop(0, QSTG // L)
        def _build(gg):
            sl = pl.ds(gg * L, L)
            osl = pl.ds(p * QSTG + gg * L, L)
            cum = qbuf[0, sl]
            first = cum
            svecs = []
            for j in range(8):
                nxt = jnp.maximum(cum, qbuf[j + 1, sl])
                svecs.append((nxt - cum) * ivecs[j])
                cum = nxt
            coef[0, osl] = first - svecs[0] * t0
            coef[1, osl] = svecs[0]
            for j in range(1, 8):
                coef[1 + j, osl] = svecs[j] - svecs[j - 1]

    def uslice(c):
        return u2.at[pl.ds(c * SCHUNK, SCHUNK), pl.ds(base, CPW)]

    def oslice(c):
        return out.at[pl.ds(c * SCHUNK, SCHUNK), pl.ds(base, CPW)]

    pltpu.async_copy(uslice(0), ub0, us0)
    pltpu.async_copy(uslice(1), ub1, us1)

    @pl.loop(0, NCH, step=2)
    def _chunks(c0):
        for b, (ub, ob, us, osm) in enumerate(
            ((ub0, ob0, us0, os0), (ub1, ob1, us1, os1))):
            c = c0 + b
            pltpu.make_async_copy(uslice(c), ub, us).wait()

            @pl.when(c >= 2)
            def _():
                pltpu.make_async_copy(ob, oslice(c), osm).wait()

            @pl.loop(0, GPW)
            def _grp(g):
                sl = pl.ds(g * L, L)
                cvecs = [coef[j, sl] for j in range(9)]
                for r in range(SCHUNK):
                    uv = ub[r, sl]
                    terms = [cvecs[0] + cvecs[1] * uv]
                    for j in range(1, 8):
                        terms.append(
                            cvecs[1 + j] * jnp.maximum(uv - tvecs[j - 1], 0.0))
                    while len(terms) > 1:
                        terms = [terms[i] + terms[i + 1]
                                 for i in range(0, len(terms) - 1, 2)] + (
                                     [terms[-1]] if len(terms) % 2 else [])
                    ob[r, sl] = jnp.maximum(terms[0], 0.0)

            @pl.when(c + 2 < NCH)
            def _():
                pltpu.async_copy(uslice(c + 2), ub, us)

            pltpu.async_copy(ob, oslice(c), osm)

    pltpu.make_async_copy(ob0, oslice(NCH - 2), os0).wait()
    pltpu.make_async_copy(ob1, oslice(NCH - 1), os1).wait()


@jax.jit
def kernel(q_curve, u, taus):
    qT = q_curve.reshape(NT, 9).T
    u2 = u.reshape(S, NT)
    dt = taus[1:] - taus[:-1]
    ivs = 1.0 / (dt + 1e-12)
    tsp = jnp.broadcast_to(taus[:8, None], (8, L)).astype(jnp.float32)
    ivb = jnp.broadcast_to(ivs[:, None], (8, L)).astype(jnp.float32)

    mesh = plsc.VectorSubcoreMesh(core_axis_name="c", subcore_axis_name="s")
    run = pl.kernel(
        _sc_body,
        out_type=jax.ShapeDtypeStruct((SSC, NT), jnp.float32),
        mesh=mesh,
        compiler_params=pltpu.CompilerParams(needs_layout_passes=False),
        scratch_types=[
            pltpu.VMEM((9, QSTG), jnp.float32),
            pltpu.VMEM((9, CPW), jnp.float32),
            pltpu.VMEM((8, L), jnp.float32),
            pltpu.VMEM((8, L), jnp.float32),
            pltpu.VMEM((SCHUNK, CPW), jnp.float32),
            pltpu.VMEM((SCHUNK, CPW), jnp.float32),
            pltpu.VMEM((SCHUNK, CPW), jnp.float32),
            pltpu.VMEM((SCHUNK, CPW), jnp.float32),
            pltpu.SemaphoreType.DMA,
            pltpu.SemaphoreType.DMA,
            pltpu.SemaphoreType.DMA,
            pltpu.SemaphoreType.DMA,
        ],
    )
    scen_sc = run(qT, u2[:SSC], tsp, ivb)

    scen_tc = pl.pallas_call(
        _tc_body,
        grid=(NT // CB,),
        in_specs=[
            pl.BlockSpec((9, CB), lambda i: (0, i)),
            pl.BlockSpec((NTC, CB), lambda i: (0, i)),
            pl.BlockSpec(memory_space=pltpu.SMEM),
        ],
        out_specs=pl.BlockSpec((NTC, CB), lambda i: (0, i)),
        out_shape=jax.ShapeDtypeStruct((NTC, NT), jnp.float32),
    )(qT, u2[SSC:], taus.astype(jnp.float32))

    scen = jnp.concatenate([scen_sc, scen_tc], axis=0)
    return scen.reshape(S, 4096, 24)


# speedup vs baseline: 1.1897x; 1.1897x over previous
import jax
import jax.numpy as jnp
from jax import lax
from jax.experimental import pallas as pl
from jax.experimental.pallas import tpu as pltpu
from jax.experimental.pallas import tpu_sc as plsc

L = 16
NW = 32
NT = 4096 * 24
S = 128
SSC = 32
CPW = NT // NW
GPW = CPW // L
SCHUNK = 4
NCH = SSC // SCHUNK
QSTG = 1024
RB = SSC
CB = 3072


def _coef_body(q2f, tsp, iv, coefout, qbuf, cbuf, tbuf, ibuf):
    nc = 2
    wid = lax.axis_index("s") * nc + lax.axis_index("c")
    base = wid * CPW

    pltpu.sync_copy(tsp, tbuf)
    pltpu.sync_copy(iv, ibuf)
    ivecs = [ibuf[j, :] for j in range(8)]
    t0 = tbuf[0, :]
    iota = lax.iota(jnp.int32, L)

    for p in range(CPW // QSTG):
        pltpu.sync_copy(q2f.at[pl.ds((base + p * QSTG) * 9, QSTG * 9)], qbuf)

        @pl.loop(0, QSTG // L)
        def _build(gg):
            osl = pl.ds(p * QSTG + gg * L, L)
            row9 = (gg * L + iota) * 9
            cum = plsc.load_gather(qbuf, [row9])
            first = cum
            svecs = []
            for j in range(8):
                nxt = jnp.maximum(cum, plsc.load_gather(qbuf, [row9 + (j + 1)]))
                svecs.append((nxt - cum) * ivecs[j])
                cum = nxt
            cbuf[0, osl] = first - svecs[0] * t0
            cbuf[1, osl] = svecs[0]
            for j in range(1, 8):
                cbuf[1 + j, osl] = svecs[j] - svecs[j - 1]

    pltpu.sync_copy(cbuf, coefout.at[:, pl.ds(base, CPW)])


def _sc_body(u2, coef, tsp, out,
             cbuf, tbuf, ub0, ub1, ob0, ob1, us0, us1, os0, os1):
    nc = 2
    wid = lax.axis_index("s") * nc + lax.axis_index("c")
    base = wid * CPW

    pltpu.sync_copy(coef.at[:, pl.ds(base, CPW)], cbuf)
    pltpu.sync_copy(tsp, tbuf)
    tvecs = [tbuf[j, :] for j in range(1, 8)]

    def uslice(c):
        return u2.at[pl.ds(c * SCHUNK, SCHUNK), pl.ds(base, CPW)]

    def oslice(c):
        return out.at[pl.ds(c * SCHUNK, SCHUNK), pl.ds(base, CPW)]

    pltpu.async_copy(uslice(0), ub0, us0)
    pltpu.async_copy(uslice(1), ub1, us1)

    @pl.loop(0, NCH, step=2)
    def _chunks(c0):
        for b, (ub, ob, us, osm) in enumerate(
            ((ub0, ob0, us0, os0), (ub1, ob1, us1, os1))):
            c = c0 + b
            pltpu.make_async_copy(uslice(c), ub, us).wait()

            @pl.when(c >= 2)
            def _():
                pltpu.make_async_copy(ob, oslice(c), osm).wait()

            @pl.loop(0, GPW)
            def _grp(g):
                sl = pl.ds(g * L, L)
                cvecs = [cbuf[j, sl] for j in range(9)]
                for r in range(SCHUNK):
                    uv = ub[r, sl]
                    terms = [cvecs[0] + cvecs[1] * uv]
                    for j in range(1, 8):
                        terms.append(
                            cvecs[1 + j] * jnp.maximum(uv - tvecs[j - 1], 0.0))
                    while len(terms) > 1:
                        terms = [terms[i] + terms[i + 1]
                                 for i in range(0, len(terms) - 1, 2)] + (
                                     [terms[-1]] if len(terms) % 2 else [])
                    ob[r, sl] = jnp.maximum(terms[0], 0.0)

            @pl.when(c + 2 < NCH)
            def _():
                pltpu.async_copy(uslice(c + 2), ub, us)

            pltpu.async_copy(ob, oslice(c), osm)

    pltpu.make_async_copy(ob0, oslice(NCH - 2), os0).wait()
    pltpu.make_async_copy(ob1, oslice(NCH - 1), os1).wait()


def _tc_body(u_ref, coef_ref, scsub_ref, taus_ref, out_ref):
    r = pl.program_id(0)

    @pl.when(r == 0)
    def _():
        out_ref[...] = scsub_ref[...]

    @pl.when(r > 0)
    def _():
        t = [taus_ref[j] for j in range(8)]
        a = coef_ref[0:1, :]
        b = coef_ref[1:2, :]
        ub = u_ref[...]
        terms = [a + b * ub]
        for j in range(1, 8):
            terms.append(coef_ref[1 + j:2 + j, :]
                         * jnp.maximum(ub - t[j], 0.0))
        while len(terms) > 1:
            terms = [terms[i] + terms[i + 1]
                     for i in range(0, len(terms) - 1, 2)] + (
                         [terms[-1]] if len(terms) % 2 else [])
        out_ref[...] = jnp.maximum(terms[0], 0.0)


@jax.jit
def kernel(q_curve, u, taus):
    q2f = q_curve.reshape(NT * 9)
    u2 = u.reshape(S, NT)
    dt = taus[1:] - taus[:-1]
    ivs = 1.0 / (dt + 1e-12)
    tsp = jnp.broadcast_to(taus[:8, None], (8, L)).astype(jnp.float32)
    ivb = jnp.broadcast_to(ivs[:, None], (8, L)).astype(jnp.float32)

    mesh = plsc.VectorSubcoreMesh(core_axis_name="c", subcore_axis_name="s")
    sc_params = pltpu.CompilerParams(needs_layout_passes=False)

    coef = pl.kernel(
        _coef_body,
        out_type=jax.ShapeDtypeStruct((9, NT), jnp.float32),
        mesh=mesh,
        compiler_params=sc_params,
        scratch_types=[
            pltpu.VMEM((QSTG * 9,), jnp.float32),
            pltpu.VMEM((9, CPW), jnp.float32),
            pltpu.VMEM((8, L), jnp.float32),
            pltpu.VMEM((8, L), jnp.float32),
        ],
    )(q2f, tsp, ivb)

    scen_sc = pl.kernel(
        _sc_body,
        out_type=jax.ShapeDtypeStruct((SSC, NT), jnp.float32),
        mesh=mesh,
        compiler_params=sc_params,
        scratch_types=[
            pltpu.VMEM((9, CPW), jnp.float32),
            pltpu.VMEM((8, L), jnp.float32),
            pltpu.VMEM((SCHUNK, CPW), jnp.float32),
            pltpu.VMEM((SCHUNK, CPW), jnp.float32),
            pltpu.VMEM((SCHUNK, CPW), jnp.float32),
            pltpu.VMEM((SCHUNK, CPW), jnp.float32),
            pltpu.SemaphoreType.DMA,
            pltpu.SemaphoreType.DMA,
            pltpu.SemaphoreType.DMA,
            pltpu.SemaphoreType.DMA,
        ],
    )(u2, coef, tsp)

    scen = pl.pallas_call(
        _tc_body,
        grid=(S // RB, NT // CB),
        in_specs=[
            pl.BlockSpec((RB, CB), lambda r, c: (r, c)),
            pl.BlockSpec((9, CB), lambda r, c: (0, c)),
            pl.BlockSpec((SSC, CB), lambda r, c: (0, c)),
            pl.BlockSpec(memory_space=pltpu.SMEM),
        ],
        out_specs=pl.BlockSpec((RB, CB), lambda r, c: (r, c)),
        out_shape=jax.ShapeDtypeStruct((S, NT), jnp.float32),
    )(u2, coef, scen_sc, taus.astype(jnp.float32))

    return scen.reshape(S, 4096, 24)
